# baseline (speedup 1.0000x reference)
import functools

import jax
import jax.numpy as jnp
from jax import lax
from jax.experimental import pallas as pl
from jax.experimental.pallas import tpu as pltpu
from jax.experimental.pallas import tpu_sc as plsc

VOCAB = 1000000
DIM = 64
BATCH = 4096
HIST = 200

NC = 2
NS = 16
NW = NC * NS
L = 16

CB = 128
NBC = BATCH // CB
NITEM = HIST * NBC
ITEMS_PER_W = NITEM // NW


@functools.partial(
    pl.kernel,
    out_type=jax.ShapeDtypeStruct((HIST, DIM, BATCH), jnp.float32),
    mesh=plsc.VectorSubcoreMesh(
        core_axis_name="c", subcore_axis_name="s", num_cores=NC, num_subcores=NS
    ),
    scratch_types=[
        *[pltpu.VMEM((CB,), jnp.int32) for _ in range(2)],
        *[pltpu.VMEM((CB,), jnp.int32) for _ in range(2)],
        *[pltpu.VMEM((CB, 2 * DIM), jnp.float32) for _ in range(2)],
        *[pltpu.VMEM((DIM, CB), jnp.float32) for _ in range(2)],
        *[pltpu.SemaphoreType.DMA for _ in range(4)],
    ],
    compiler_params=pltpu.CompilerParams(needs_layout_passes=False),
)
def _gather_kernel(idx_hbm, table2_hbm, out_hbm, *bufs):
    idx_v = list(bufs[0:2])
    pidx_v = list(bufs[2:4])
    pair_v = list(bufs[4:6])
    tr_v = list(bufs[6:8])
    gsem = list(bufs[8:10])
    osem = list(bufs[10:12])

    wid = lax.axis_index("s") * NC + lax.axis_index("c")
    q0 = wid * ITEMS_PER_W

    def load_idx(i, b):
        q = q0 + i
        t = q // NBC
        bc = q % NBC
        pltpu.sync_copy(idx_hbm.at[t, pl.ds(bc * CB, CB)], idx_v[b])

        def mk_pidx(g, c2):
            v = idx_v[b][pl.ds(g * L, L)]
            pidx_v[b][pl.ds(g * L, L)] = lax.shift_right_logical(v, 1)
            return c2

        lax.fori_loop(0, CB // L, mk_pidx, 0)

    def start_gather(b):
        pltpu.async_copy(table2_hbm.at[pidx_v[b]], pair_v[b], gsem[b])

    def wait_gather(b):
        pltpu.make_async_copy(table2_hbm.at[pidx_v[b]], pair_v[b], gsem[b]).wait()

    def out_slice(i):
        q = q0 + i
        t = q // NBC
        bc = q % NBC
        return out_hbm.at[t, :, pl.ds(bc * CB, CB)]

    def start_store(i, b):
        pltpu.async_copy(tr_v[b], out_slice(i), osem[b])

    def wait_store(i, b):
        pltpu.make_async_copy(tr_v[b], out_slice(i), osem[b]).wait()

    def transpose(b):
        def grp(g, c2):
            jvec = g * L + lax.iota(jnp.int32, L)
            raw = idx_v[b][pl.ds(g * L, L)]
            colbase = jnp.bitwise_and(raw, 1) * DIM

            def col(c8, c3):
                for u in range(8):
                    c = c8 * 8 + u
                    vals = plsc.load_gather(pair_v[b], [jvec, colbase + c])
                    tr_v[b][c, pl.ds(g * L, L)] = vals
                return c3

            lax.fori_loop(0, DIM // 8, col, 0)
            return c2

        lax.fori_loop(0, CB // L, grp, 0)

    load_idx(0, 0)
    start_gather(0)

    def outer(h, carry):
        for b in range(2):
            i = h * 2 + b
            bn = 1 - b
            wait_gather(b)

            @pl.when(i + 1 < ITEMS_PER_W)
            def _():
                load_idx(i + 1, bn)

                @pl.when(i + 1 >= 2)
                def _():
                    wait_store(i - 1, bn)

                start_gather(bn)

            transpose(b)
            start_store(i, b)
        return carry

    lax.fori_loop(0, ITEMS_PER_W // 2, outer, 0)

    wait_store(ITEMS_PER_W - 2, 0)
    wait_store(ITEMS_PER_W - 1, 1)


def kernel(inputs, embed_weight):
    idx_t = inputs.T
    table2 = embed_weight.reshape((VOCAB // 2, 2 * DIM))
    out_t = _gather_kernel(idx_t, table2)
    return jnp.transpose(out_t, (2, 0, 1))
